# baseline (speedup 1.0000x reference)
import functools

import jax
import jax.numpy as jnp
from jax import lax
from jax.experimental import pallas as pl
from jax.experimental.pallas import tpu as pltpu
from jax.experimental.pallas import tpu_sc as plsc

B = 4096
D = 128
K = 20
T = 128
NNODES = 100000
FLAT_ROWS = (NNODES * K) // 128

_info = plsc.get_sparse_core_info()
NC = _info.num_cores
NS = _info.num_subcores
NW = NC * NS
EPW = B // NW
CH = 128
NCHUNK = (EPW * K) // CH
NPASS = 2
PP = EPW // NPASS
NBUF = 4

_mesh = plsc.VectorSubcoreMesh(core_axis_name="c", subcore_axis_name="s")


@functools.partial(
    pl.kernel,
    mesh=_mesh,
    compiler_params=pltpu.CompilerParams(needs_layout_passes=False),
    out_type=[
        jax.ShapeDtypeStruct((B, D), jnp.float32),
        jax.ShapeDtypeStruct((B, D), jnp.float32),
        jax.ShapeDtypeStruct((B * K, D), jnp.float32),
        jax.ShapeDtypeStruct((B * 128,), jnp.float32),
    ],
    scratch_types=[
        pltpu.VMEM((NCHUNK + 2, CH), jnp.int32),
        pltpu.VMEM((EPW,), jnp.float32),
        pltpu.VMEM((EPW,), jnp.int32),
        pltpu.VMEM((EPW,), jnp.int32),
        pltpu.VMEM((EPW,), jnp.int32),
        pltpu.VMEM((EPW,), jnp.int32),
        pltpu.VMEM((EPW,), jnp.int32),
        pltpu.VMEM((PP, 128), jnp.int32),
        pltpu.VMEM((PP, 128), jnp.int32),
        pltpu.VMEM((PP, 128), jnp.int32),
        pltpu.VMEM((PP, 128), jnp.int32),
        pltpu.VMEM((PP * 128,), jnp.float32),
    ] + [pltpu.VMEM((CH, D), jnp.float32) for _ in range(NBUF)]
      + [pltpu.SemaphoreType.DMA for _ in range(4 + NBUF)],
)
def _sc_gather(src_hbm, dst_hbm, ts_hbm, adjf_hbm, nfeat_hbm,
               srcf_out, dstf_out, nbf_out, dt_out,
               idxall, ts_v, r0_v, r1_v, r0t_v, r1t_v, off_v,
               nraw0, nraw1, traw0, traw1, dtbuf, *bufs_and_sems):
    bufs = bufs_and_sems[:NBUF]
    sem_n0, sem_n1, sem_t0, sem_t1 = bufs_and_sems[NBUF:NBUF + 4]
    sems = bufs_and_sems[NBUF + 4:]
    wid = lax.axis_index("s") * NC + lax.axis_index("c")
    base = wid * EPW
    pltpu.sync_copy(src_hbm.at[pl.ds(base, EPW)], idxall.at[0])
    pltpu.sync_copy(dst_hbm.at[pl.ds(base, EPW)], idxall.at[1])
    pltpu.sync_copy(ts_hbm.at[pl.ds(base, EPW)], ts_v)
    for m in range(EPW // 16):
        sl = pl.ds(m * 16, 16)
        a = idxall[0, sl] * 20
        r0 = lax.shift_right_logical(a, 7)
        r1 = jnp.minimum(r0 + 1, FLAT_ROWS - 1)
        r0_v[sl] = r0
        r1_v[sl] = r1
        r0t_v[sl] = r0 + FLAT_ROWS
        r1t_v[sl] = r1 + FLAT_ROWS
        off_v[sl] = lax.bitwise_and(a, 127)
    lane = lax.iota(jnp.int32, 16)
    for h in range(NPASS):
        e0 = h * PP
        c_n0 = pltpu.async_copy(
            adjf_hbm.at[r0_v.at[pl.ds(e0, PP)]], nraw0, sem_n0)
        c_n1 = pltpu.async_copy(
            adjf_hbm.at[r1_v.at[pl.ds(e0, PP)]], nraw1, sem_n1)
        c_t0 = pltpu.async_copy(
            adjf_hbm.at[r0t_v.at[pl.ds(e0, PP)]], traw0, sem_t0)
        c_t1 = pltpu.async_copy(
            adjf_hbm.at[r1t_v.at[pl.ds(e0, PP)]], traw1, sem_t1)
        c_n0.wait()
        c_n1.wait()
        c_t0.wait()
        c_t1.wait()
        for g in range(PP // 16):
            evl = lane + (16 * g)
            ev = evl + e0
            off = off_v[pl.ds(e0 + g * 16, 16)]
            ts = ts_v[pl.ds(e0 + g * 16, 16)]
            for k in range(K):
                c = off + k
                hi = c >= 128
                cm = lax.bitwise_and(c, 127)
                n0 = plsc.load_gather(nraw0, [evl, cm])
                n1 = plsc.load_gather(nraw1, [evl, cm])
                t0 = plsc.load_gather(traw0, [evl, cm])
                t1 = plsc.load_gather(traw1, [evl, cm])
                et = plsc.bitcast(jnp.where(hi, t1, t0), jnp.float32)
                pos = ev * K + k + 2 * CH
                plsc.store_scatter(
                    idxall, [lax.shift_right_logical(pos, 7),
                             lax.bitwise_and(pos, 127)],
                    jnp.where(hi, n1, n0))
                plsc.store_scatter(dtbuf, [evl * 128 + k], ts - et)
        pltpu.sync_copy(
            dtbuf, dt_out.at[pl.ds((base + e0) * 128, PP * 128)])
    copies = [None] * NBUF
    nbase = wid * NCHUNK * CH

    def _drain(j):
        s = j % NBUF
        copies[s].wait()
        if j == 0:
            dst = srcf_out.at[pl.ds(base, EPW)]
        elif j == 1:
            dst = dstf_out.at[pl.ds(base, EPW)]
        else:
            dst = nbf_out.at[pl.ds(nbase + (j - 2) * CH, CH)]
        pltpu.sync_copy(bufs[s], dst)

    for j in range(NCHUNK + 2):
        s = j % NBUF
        if copies[s] is not None:
            _drain(j - NBUF)
        copies[s] = pltpu.async_copy(nfeat_hbm.at[idxall.at[j]], bufs[s],
                                     sems[s])
    for j in range(NCHUNK + 2 - NBUF, NCHUNK + 2):
        _drain(j)


R = 128

_INV2PI = 0.15915494309189535
_MAGIC = 12582912.0
_COSC = (0.9999999604622878, -19.739082387888594, 64.93140197980571,
         -85.29869317227555, 58.90142703994175, -21.23133574147967)


def _fast_cos(x):
    y = x * _INV2PI
    n = (y + _MAGIC) - _MAGIC
    f = y - n
    v = f * f
    p = jnp.float32(_COSC[5])
    for c in _COSC[4::-1]:
        p = p * v + jnp.float32(c)
    return p


def _tc_body(srcf, dstf, nbf, dt, tw, tb, out):
    w = tw[...]
    b = tb[...]
    const_emb = jnp.broadcast_to(_fast_cos(b), (R, T))
    out[:, 0:D] = srcf[...]
    out[:, D:D + T] = const_emb
    out[:, D + T:2 * D + T] = dstf[...]
    out[:, 2 * D + T:2 * D + 2 * T] = const_emb
    out[:, 2 * D + 2 * T:2 * D + 2 * T + K * D] = nbf[...]
    dt_v = dt[...]
    base = 2 * D + 2 * T + K * D
    for k in range(K):
        dk = dt_v[:, k:k + 1]
        out[:, base + k * T:base + (k + 1) * T] = _fast_cos(dk * w + b)


def _tc_assemble(srcf, dstf, nbf2d, dt2d, tw2d, tb2d):
    ncols = 2 * D + 2 * T + K * D + K * T
    return pl.pallas_call(
        _tc_body,
        grid=(B // R,),
        in_specs=[
            pl.BlockSpec((R, D), lambda i: (i, 0)),
            pl.BlockSpec((R, D), lambda i: (i, 0)),
            pl.BlockSpec((R, K * D), lambda i: (i, 0)),
            pl.BlockSpec((R, 128), lambda i: (i, 0)),
            pl.BlockSpec((1, T), lambda i: (0, 0)),
            pl.BlockSpec((1, T), lambda i: (0, 0)),
        ],
        out_specs=pl.BlockSpec((R, ncols), lambda i: (i, 0)),
        out_shape=jax.ShapeDtypeStruct((B, ncols), jnp.float32),
    )(srcf, dstf, nbf2d, dt2d, tw2d, tb2d)


def kernel(source_nodes, destination_nodes, source_timestamps,
           destination_timestamps, n_neighbors, node_features, time_w,
           time_b, neighbor_table, neighbor_edge_times):
    src = source_nodes.astype(jnp.int32)
    dst = destination_nodes.astype(jnp.int32)
    adj = jnp.concatenate(
        [neighbor_table.astype(jnp.int32),
         lax.bitcast_convert_type(neighbor_edge_times, jnp.int32)],
        axis=0).reshape(2 * FLAT_ROWS, 128)
    srcf, dstf, nbf, dts = _sc_gather(
        src, dst, source_timestamps, adj, node_features)
    out = _tc_assemble(
        srcf, dstf, nbf.reshape(B, K * D), dts.reshape(B, 128),
        time_w.reshape(1, T), time_b.reshape(1, T))
    return out


# speedup vs baseline: 1.1783x; 1.1783x over previous
import functools

import jax
import jax.numpy as jnp
from jax import lax
from jax.experimental import pallas as pl
from jax.experimental.pallas import tpu as pltpu
from jax.experimental.pallas import tpu_sc as plsc

B = 4096
D = 128
K = 20
T = 128
NNODES = 100000
FLAT_ROWS = (NNODES * K) // 128

_info = plsc.get_sparse_core_info()
NC = _info.num_cores
NS = _info.num_subcores
NW = NC * NS
EPW = B // NW
CH = 128
NCHUNK = (EPW * K) // CH
NPASS = 2
PP = EPW // NPASS
NBUF = 4

_mesh = plsc.VectorSubcoreMesh(core_axis_name="c", subcore_axis_name="s")


@functools.partial(
    pl.kernel,
    mesh=_mesh,
    compiler_params=pltpu.CompilerParams(needs_layout_passes=False),
    out_type=[
        jax.ShapeDtypeStruct((B, D), jnp.float32),
        jax.ShapeDtypeStruct((B, D), jnp.float32),
        jax.ShapeDtypeStruct((B * K, D), jnp.float32),
        jax.ShapeDtypeStruct((B * 128,), jnp.float32),
    ],
    scratch_types=[
        pltpu.VMEM((NCHUNK + 2, CH), jnp.int32),
        pltpu.VMEM((EPW,), jnp.float32),
        pltpu.VMEM((EPW,), jnp.int32),
        pltpu.VMEM((EPW,), jnp.int32),
        pltpu.VMEM((EPW,), jnp.int32),
        pltpu.VMEM((PP, 128), jnp.int32),
        pltpu.VMEM((PP, 128), jnp.int32),
        pltpu.VMEM((PP, 128), jnp.float32),
        pltpu.VMEM((PP, 128), jnp.float32),
        pltpu.VMEM((PP * 128,), jnp.float32),
    ] + [pltpu.VMEM((CH, D), jnp.float32) for _ in range(NBUF)]
      + [pltpu.SemaphoreType.DMA for _ in range(4 + NBUF)],
)
def _sc_gather(src_hbm, dst_hbm, ts_hbm, ntabf_hbm, ntimef_hbm, nfeat_hbm,
               srcf_out, dstf_out, nbf_out, dt_out,
               idxall, ts_v, r0_v, r1_v, off_v,
               nraw0, nraw1, traw0, traw1, dtbuf, *bufs_and_sems):
    bufs = bufs_and_sems[:NBUF]
    sem_n0, sem_n1, sem_t0, sem_t1 = bufs_and_sems[NBUF:NBUF + 4]
    sems = bufs_and_sems[NBUF + 4:]
    wid = lax.axis_index("s") * NC + lax.axis_index("c")
    base = wid * EPW
    pltpu.sync_copy(src_hbm.at[pl.ds(base, EPW)], idxall.at[0])
    pltpu.sync_copy(dst_hbm.at[pl.ds(base, EPW)], idxall.at[1])
    pltpu.sync_copy(ts_hbm.at[pl.ds(base, EPW)], ts_v)
    for m in range(EPW // 16):
        sl = pl.ds(m * 16, 16)
        a = idxall[0, sl] * 20
        r0 = lax.shift_right_logical(a, 7)
        r0_v[sl] = r0
        r1_v[sl] = jnp.minimum(r0 + 1, FLAT_ROWS - 1)
        off_v[sl] = lax.bitwise_and(a, 127)
    lane = lax.iota(jnp.int32, 16)
    for h in range(NPASS):
        e0 = h * PP
        c_n0 = pltpu.async_copy(
            ntabf_hbm.at[r0_v.at[pl.ds(e0, PP)]], nraw0, sem_n0)
        c_n1 = pltpu.async_copy(
            ntabf_hbm.at[r1_v.at[pl.ds(e0, PP)]], nraw1, sem_n1)
        c_t0 = pltpu.async_copy(
            ntimef_hbm.at[r0_v.at[pl.ds(e0, PP)]], traw0, sem_t0)
        c_t1 = pltpu.async_copy(
            ntimef_hbm.at[r1_v.at[pl.ds(e0, PP)]], traw1, sem_t1)
        c_n0.wait()
        c_n1.wait()
        c_t0.wait()
        c_t1.wait()
        for g in range(PP // 16):
            evl = lane + (16 * g)
            ev = evl + e0
            off = off_v[pl.ds(e0 + g * 16, 16)]
            ts = ts_v[pl.ds(e0 + g * 16, 16)]
            for k in range(K):
                c = off + k
                hi = c >= 128
                cm = lax.bitwise_and(c, 127)
                n0 = plsc.load_gather(nraw0, [evl, cm])
                n1 = plsc.load_gather(nraw1, [evl, cm])
                t0 = plsc.load_gather(traw0, [evl, cm])
                t1 = plsc.load_gather(traw1, [evl, cm])
                pos = ev * K + k + 2 * CH
                plsc.store_scatter(
                    idxall, [lax.shift_right_logical(pos, 7),
                             lax.bitwise_and(pos, 127)],
                    lax.bitwise_xor(jnp.where(hi, n1, n0), 1))
                plsc.store_scatter(dtbuf, [evl * 128 + k],
                                   ts + jnp.where(hi, t1, t0))
        pltpu.sync_copy(
            dtbuf, dt_out.at[pl.ds((base + e0) * 128, PP * 128)])
    copies = [None] * NBUF
    nbase = wid * NCHUNK * CH

    def _drain(j):
        s = j % NBUF
        copies[s].wait()
        if j == 0:
            dst = srcf_out.at[pl.ds(base, EPW)]
        elif j == 1:
            dst = dstf_out.at[pl.ds(base, EPW)]
        else:
            dst = nbf_out.at[pl.ds(nbase + (j - 2) * CH, CH)]
        pltpu.sync_copy(bufs[s], dst)

    for j in range(NCHUNK + 2):
        s = j % NBUF
        if copies[s] is not None:
            _drain(j - NBUF)
        copies[s] = pltpu.async_copy(nfeat_hbm.at[idxall.at[j]], bufs[s],
                                     sems[s])
    for j in range(NCHUNK + 2 - NBUF, NCHUNK + 2):
        _drain(j)


R = 128

_INV2PI = 0.15915494309189535
_MAGIC = 12582912.0
_COSC = (0.9999999604622878, -19.739082387888594, 64.93140197980571,
         -85.29869317227555, 58.90142703994175, -21.23133574147967)


def _fast_cos(x):
    y = x * _INV2PI
    n = (y + _MAGIC) - _MAGIC
    f = y - n
    v = f * f
    p = jnp.float32(_COSC[5])
    for c in _COSC[4::-1]:
        p = p * v + jnp.float32(c)
    return p


def _tc_body(srcf, dstf, nbf, dt, tw, tb, out):
    w = tw[...]
    b = tb[...]
    const_emb = jnp.broadcast_to(_fast_cos(b), (R, T))
    out[:, 0:D] = srcf[...]
    out[:, D:D + T] = const_emb
    out[:, D + T:2 * D + T] = dstf[...]
    out[:, 2 * D + T:2 * D + 2 * T] = const_emb
    out[:, 2 * D + 2 * T:2 * D + 2 * T + K * D] = nbf[...]
    dt_v = dt[...]
    base = 2 * D + 2 * T + K * D
    for k in range(K):
        dk = dt_v[:, k:k + 1]
        out[:, base + k * T:base + (k + 1) * T] = _fast_cos(dk * w + b)


def _tc_assemble(srcf, dstf, nbf2d, dt2d, tw2d, tb2d):
    ncols = 2 * D + 2 * T + K * D + K * T
    return pl.pallas_call(
        _tc_body,
        grid=(B // R,),
        in_specs=[
            pl.BlockSpec((R, D), lambda i: (i, 0)),
            pl.BlockSpec((R, D), lambda i: (i, 0)),
            pl.BlockSpec((R, K * D), lambda i: (i, 0)),
            pl.BlockSpec((R, 128), lambda i: (i, 0)),
            pl.BlockSpec((1, T), lambda i: (0, 0)),
            pl.BlockSpec((1, T), lambda i: (0, 0)),
        ],
        out_specs=pl.BlockSpec((R, ncols), lambda i: (i, 0)),
        out_shape=jax.ShapeDtypeStruct((B, ncols), jnp.float32),
    )(srcf, dstf, nbf2d, dt2d, tw2d, tb2d)


def kernel(source_nodes, destination_nodes, source_timestamps,
           destination_timestamps, n_neighbors, node_features, time_w,
           time_b, neighbor_table, neighbor_edge_times):
    src = source_nodes.astype(jnp.int32)
    dst = destination_nodes.astype(jnp.int32)
    ntab_flat = lax.bitwise_xor(
        neighbor_table.astype(jnp.int32), 1).reshape(FLAT_ROWS, 128)
    ntime_flat = (-neighbor_edge_times).reshape(FLAT_ROWS, 128)
    srcf, dstf, nbf, dts = _sc_gather(
        src, dst, source_timestamps, ntab_flat, ntime_flat, node_features)
    out = _tc_assemble(
        srcf, dstf, nbf.reshape(B, K * D), dts.reshape(B, 128),
        time_w.reshape(1, T), time_b.reshape(1, T))
    return out


# speedup vs baseline: 1.2493x; 1.0602x over previous
import functools

import jax
import jax.numpy as jnp
from jax import lax
from jax.experimental import pallas as pl
from jax.experimental.pallas import tpu as pltpu
from jax.experimental.pallas import tpu_sc as plsc

B = 4096
D = 128
K = 20
T = 128
NNODES = 100000
FLAT_ROWS = (NNODES * K) // 128

_info = plsc.get_sparse_core_info()
NC = _info.num_cores
NS = _info.num_subcores
NW = NC * NS
EPW = B // NW
CH = 128
NCHUNK = (EPW * K) // CH
NPASS = 2
PP = EPW // NPASS
NBUF = 4

_mesh = plsc.VectorSubcoreMesh(core_axis_name="c", subcore_axis_name="s")


@functools.partial(
    pl.kernel,
    mesh=_mesh,
    compiler_params=pltpu.CompilerParams(needs_layout_passes=False),
    out_type=[
        jax.ShapeDtypeStruct((B, D), jnp.float32),
        jax.ShapeDtypeStruct((B, D), jnp.float32),
        jax.ShapeDtypeStruct((B * K, D), jnp.float32),
        jax.ShapeDtypeStruct((B * 128,), jnp.float32),
    ],
    scratch_types=[
        pltpu.VMEM((NCHUNK + 2, CH), jnp.int32),
        pltpu.VMEM((EPW,), jnp.float32),
        pltpu.VMEM((EPW,), jnp.int32),
        pltpu.VMEM((EPW,), jnp.int32),
        pltpu.VMEM((EPW,), jnp.int32),
        pltpu.VMEM((PP, 128), jnp.int32),
        pltpu.VMEM((PP, 128), jnp.int32),
        pltpu.VMEM((PP, 128), jnp.float32),
        pltpu.VMEM((PP, 128), jnp.float32),
        pltpu.VMEM((PP * 128,), jnp.float32),
    ] + [pltpu.VMEM((CH, D), jnp.float32) for _ in range(NBUF)]
      + [pltpu.SemaphoreType.DMA for _ in range(4 + NBUF)],
)
def _sc_gather(src_hbm, dst_hbm, ts_hbm, ntabf_hbm, ntimef_hbm, nfeat_hbm,
               srcf_out, dstf_out, nbf_out, dt_out,
               idxall, ts_v, r0_v, r1_v, off_v,
               nraw0, nraw1, traw0, traw1, dtbuf, *bufs_and_sems):
    bufs = bufs_and_sems[:NBUF]
    sem_n0, sem_n1, sem_t0, sem_t1 = bufs_and_sems[NBUF:NBUF + 4]
    sems = bufs_and_sems[NBUF + 4:]
    wid = lax.axis_index("s") * NC + lax.axis_index("c")
    base = wid * EPW
    pltpu.sync_copy(src_hbm.at[pl.ds(base, EPW)], idxall.at[0])
    pltpu.sync_copy(dst_hbm.at[pl.ds(base, EPW)], idxall.at[1])
    pltpu.sync_copy(ts_hbm.at[pl.ds(base, EPW)], ts_v)
    copies = [None] * NBUF
    for j in (0, 1):
        copies[j] = pltpu.async_copy(nfeat_hbm.at[idxall.at[j]], bufs[j],
                                     sems[j])
    for m in range(EPW // 16):
        sl = pl.ds(m * 16, 16)
        a = idxall[0, sl] * 20
        r0 = lax.shift_right_logical(a, 7)
        r0_v[sl] = r0
        r1_v[sl] = jnp.minimum(r0 + 1, FLAT_ROWS - 1)
        off_v[sl] = lax.bitwise_and(a, 127)
    lane = lax.iota(jnp.int32, 16)
    for h in range(NPASS):
        e0 = h * PP
        c_n0 = pltpu.async_copy(
            ntabf_hbm.at[r0_v.at[pl.ds(e0, PP)]], nraw0, sem_n0)
        c_n1 = pltpu.async_copy(
            ntabf_hbm.at[r1_v.at[pl.ds(e0, PP)]], nraw1, sem_n1)
        c_t0 = pltpu.async_copy(
            ntimef_hbm.at[r0_v.at[pl.ds(e0, PP)]], traw0, sem_t0)
        c_t1 = pltpu.async_copy(
            ntimef_hbm.at[r1_v.at[pl.ds(e0, PP)]], traw1, sem_t1)
        c_n0.wait()
        c_n1.wait()
        c_t0.wait()
        c_t1.wait()
        for g in range(PP // 16):
            evl = lane + (16 * g)
            ev = evl + e0
            off = off_v[pl.ds(e0 + g * 16, 16)]
            ts = ts_v[pl.ds(e0 + g * 16, 16)]
            for k in range(K):
                c = off + k
                hi = c >= 128
                cm = lax.bitwise_and(c, 127)
                n0 = plsc.load_gather(nraw0, [evl, cm])
                n1 = plsc.load_gather(nraw1, [evl, cm])
                t0 = plsc.load_gather(traw0, [evl, cm])
                t1 = plsc.load_gather(traw1, [evl, cm])
                pos = ev * K + k + 2 * CH
                plsc.store_scatter(
                    idxall, [lax.shift_right_logical(pos, 7),
                             lax.bitwise_and(pos, 127)],
                    jnp.where(hi, n1, n0))
                plsc.store_scatter(dtbuf, [evl * 128 + k],
                                   ts - jnp.where(hi, t1, t0))
        pltpu.sync_copy(
            dtbuf, dt_out.at[pl.ds((base + e0) * 128, PP * 128)])
    nbase = wid * NCHUNK * CH

    def _drain(j):
        s = j % NBUF
        copies[s].wait()
        if j == 0:
            dst = srcf_out.at[pl.ds(base, EPW)]
        elif j == 1:
            dst = dstf_out.at[pl.ds(base, EPW)]
        else:
            dst = nbf_out.at[pl.ds(nbase + (j - 2) * CH, CH)]
        pltpu.sync_copy(bufs[s], dst)

    for j in range(2, NCHUNK + 2):
        s = j % NBUF
        if copies[s] is not None:
            _drain(j - NBUF)
        copies[s] = pltpu.async_copy(nfeat_hbm.at[idxall.at[j]], bufs[s],
                                     sems[s])
    for j in range(NCHUNK + 2 - NBUF, NCHUNK + 2):
        _drain(j)


R = 256

_INV2PI = 0.15915494309189535
_MAGIC = 12582912.0
_COSC = (0.9999999604622878, -19.739082387888594, 64.93140197980571,
         -85.29869317227555, 58.90142703994175, -21.23133574147967)


def _fast_cos(x):
    y = x * _INV2PI
    n = (y + _MAGIC) - _MAGIC
    f = y - n
    v = f * f
    p = jnp.float32(_COSC[5])
    for c in _COSC[4::-1]:
        p = p * v + jnp.float32(c)
    return p


def _tc_body(srcf, dstf, nbf, dt, tw, tb, out):
    w = tw[...]
    b = tb[...]
    const_emb = jnp.broadcast_to(_fast_cos(b), (R, T))
    out[:, 0:D] = srcf[...]
    out[:, D:D + T] = const_emb
    out[:, D + T:2 * D + T] = dstf[...]
    out[:, 2 * D + T:2 * D + 2 * T] = const_emb
    out[:, 2 * D + 2 * T:2 * D + 2 * T + K * D] = nbf[...]
    dt_v = dt[...]
    base = 2 * D + 2 * T + K * D
    for k in range(K):
        dk = dt_v[:, k:k + 1]
        out[:, base + k * T:base + (k + 1) * T] = _fast_cos(dk * w + b)


def _tc_assemble(srcf, dstf, nbf2d, dt2d, tw2d, tb2d):
    ncols = 2 * D + 2 * T + K * D + K * T
    return pl.pallas_call(
        _tc_body,
        grid=(B // R,),
        in_specs=[
            pl.BlockSpec((R, D), lambda i: (i, 0)),
            pl.BlockSpec((R, D), lambda i: (i, 0)),
            pl.BlockSpec((R, K * D), lambda i: (i, 0)),
            pl.BlockSpec((R, 128), lambda i: (i, 0)),
            pl.BlockSpec((1, T), lambda i: (0, 0)),
            pl.BlockSpec((1, T), lambda i: (0, 0)),
        ],
        out_specs=pl.BlockSpec((R, ncols), lambda i: (i, 0)),
        out_shape=jax.ShapeDtypeStruct((B, ncols), jnp.float32),
    )(srcf, dstf, nbf2d, dt2d, tw2d, tb2d)


def kernel(source_nodes, destination_nodes, source_timestamps,
           destination_timestamps, n_neighbors, node_features, time_w,
           time_b, neighbor_table, neighbor_edge_times):
    src = source_nodes.astype(jnp.int32)
    dst = destination_nodes.astype(jnp.int32)
    ntab_flat = neighbor_table.astype(jnp.int32).reshape(FLAT_ROWS, 128)
    ntime_flat = neighbor_edge_times.reshape(FLAT_ROWS, 128)
    srcf, dstf, nbf, dts = _sc_gather(
        src, dst, source_timestamps, ntab_flat, ntime_flat, node_features)
    out = _tc_assemble(
        srcf, dstf, nbf.reshape(B, K * D), dts.reshape(B, 128),
        time_w.reshape(1, T), time_b.reshape(1, T))
    return out
